# baseline (speedup 1.0000x reference)
import jax
import jax.numpy as jnp
from jax.experimental import pallas as pl

_N = 1000
_NC = 10
_K = _N + (_N * _N - _N - 1) // 2
_LAMBDA_SPEC = 0.05
_EPS = 1e-8


def _spectral_loss_kernel(p_ref, u_ref, out_ref):
    pt = p_ref[:].T
    px_r, py_r, pz_r = pt[0:1, :], pt[1:2, :], pt[2:3, :]
    px_c = p_ref[:, 0:1]
    py_c = p_ref[:, 1:2]
    pz_c = p_ref[:, 2:3]

    dx = px_c - px_r
    dy = py_c - py_r
    dz = pz_c - pz_r
    sq = dx * dx + dy * dy + dz * dz

    bits = jax.lax.bitcast_convert_type(sq, jnp.int32)
    res = jnp.int32(0)
    for b in range(30, -1, -1):
        trial = res | jnp.int32(1 << b)
        cnt = -jnp.sum(jax.lax.shift_right_arithmetic(bits - trial, 31))
        res = jnp.where(cnt <= _K, trial, res)
    sigma_sq = jax.lax.bitcast_convert_type(res, jnp.float32)
    sigma = jnp.sqrt(sigma_sq)
    denom = 2.0 * sigma * sigma + _EPS

    rows = jax.lax.broadcasted_iota(jnp.int32, (_N, _N), 0)
    cols = jax.lax.broadcasted_iota(jnp.int32, (_N, _N), 1)
    w = jnp.where(rows != cols, jnp.exp(-sq / denom), 0.0)

    d = jnp.sum(w, axis=1, keepdims=True)
    dinv = 1.0 / (jnp.sqrt(d) + _EPS)
    u = u_ref[:] * dinv
    v = jnp.dot(w, u, preferred_element_type=jnp.float32)
    s2 = jnp.sum(v * u)
    s1 = jnp.sum(u_ref[:] * u_ref[:])
    total = (s1 - s2) / _NC
    loss = _LAMBDA_SPEC * total / (_N * _N)
    out_ref[:, :] = jnp.full((1, 1), loss, dtype=jnp.float32)


@jax.jit
def kernel(points, outputs):
    out = pl.pallas_call(
        _spectral_loss_kernel,
        out_shape=jax.ShapeDtypeStruct((1, 1), jnp.float32),
    )(points, outputs)
    return out[0, 0]


# speedup vs baseline: 1.9737x; 1.9737x over previous
import jax
import jax.numpy as jnp
from jax.experimental import pallas as pl
from jax.experimental.pallas import tpu as pltpu

_N = 1000
_NP = 1024
_NC = 10
_NCP = 128
_K = _N + (_N * _N - _N - 1) // 2
_LAMBDA_SPEC = 0.05
_EPS = 1e-8


def _spectral_loss_kernel(p_ref, u_ref, out_ref, col_s, row_s, u_s):
    col_s[:] = jnp.zeros((_NP, 8), jnp.float32)
    row_s[:] = jnp.zeros((8, _NP), jnp.float32)
    u_s[:] = jnp.zeros((_NP, _NCP), jnp.float32)
    col_s[0:_N, 0:3] = p_ref[:]
    row_s[0:3, 0:_N] = p_ref[:].T
    u_s[0:_N, 0:_NC] = u_ref[:]

    dx = col_s[:, 0:1] - row_s[0:1, :]
    dy = col_s[:, 1:2] - row_s[1:2, :]
    dz = col_s[:, 2:3] - row_s[2:3, :]
    sq = dx * dx + dy * dy + dz * dz

    rows = jax.lax.broadcasted_iota(jnp.int32, (_NP, _NP), 0)
    cols = jax.lax.broadcasted_iota(jnp.int32, (_NP, _NP), 1)
    valid = (rows < _N) & (cols < _N)

    bits = jax.lax.bitcast_convert_type(sq, jnp.int32)
    bits = jnp.where(valid, bits, jnp.int32(0x7FFFFFFF))
    res = jnp.int32(0)
    for b in range(30, -1, -1):
        trial = res | jnp.int32(1 << b)
        cnt = -jnp.sum(jax.lax.shift_right_arithmetic(bits - trial, 31))
        res = jnp.where(cnt <= _K, trial, res)
    sigma_sq = jax.lax.bitcast_convert_type(res, jnp.float32)
    sigma = jnp.sqrt(sigma_sq)
    denom = 2.0 * sigma * sigma + _EPS

    mask_w = valid & (rows != cols)
    w = jnp.where(mask_w, jnp.exp(-sq / denom), 0.0)

    d = jnp.sum(w, axis=1, keepdims=True)
    dinv = 1.0 / (jnp.sqrt(d) + _EPS)
    u = u_s[:] * dinv
    v = jnp.dot(w, u, preferred_element_type=jnp.float32)
    s2 = jnp.sum(v * u)
    s1 = jnp.sum(u_s[:] * u_s[:])
    total = (s1 - s2) / _NC
    loss = _LAMBDA_SPEC * total / (_N * _N)
    out_ref[:, :] = jnp.full((1, 1), loss, dtype=jnp.float32)


@jax.jit
def kernel(points, outputs):
    out = pl.pallas_call(
        _spectral_loss_kernel,
        out_shape=jax.ShapeDtypeStruct((1, 1), jnp.float32),
        scratch_shapes=[
            pltpu.VMEM((_NP, 8), jnp.float32),
            pltpu.VMEM((8, _NP), jnp.float32),
            pltpu.VMEM((_NP, _NCP), jnp.float32),
        ],
    )(points, outputs)
    return out[0, 0]
